# baseline (speedup 1.0000x reference)
import functools

import jax
import jax.numpy as jnp
from jax import lax
from jax.experimental import pallas as pl
from jax.experimental.pallas import tpu as pltpu
from jax.experimental.pallas import tpu_sc as plsc


_BM = 400


def _gcn_layer1_fused(adj, x, W1, b, Wn):
    n = adj.shape[0]
    dh = W1.shape[1]

    def body(adj_ref, x_ref, w1_ref, b_ref, wn_ref, o_ref, t_scr):
        @pl.when(pl.program_id(0) == 0)
        def _():
            t_scr[...] = jnp.dot(x_ref[...], w1_ref[...],
                                 preferred_element_type=jnp.float32)

        acc = jnp.dot(adj_ref[...], t_scr[...],
                      preferred_element_type=jnp.float32)
        h = jnp.maximum(acc + b_ref[...], 0.0)
        o_ref[...] = jnp.dot(h, wn_ref[...],
                             preferred_element_type=jnp.float32)

    return pl.pallas_call(
        body,
        grid=(n // _BM,),
        in_specs=[
            pl.BlockSpec((_BM, n), lambda i: (i, 0)),
            pl.BlockSpec(x.shape, lambda i: (0, 0)),
            pl.BlockSpec(W1.shape, lambda i: (0, 0)),
            pl.BlockSpec((1, b.shape[1]), lambda i: (0, 0)),
            pl.BlockSpec(Wn.shape, lambda i: (0, 0)),
        ],
        out_specs=pl.BlockSpec((_BM, Wn.shape[1]), lambda i: (i, 0)),
        out_shape=jax.ShapeDtypeStruct((n, Wn.shape[1]), jnp.float32),
        scratch_shapes=[pltpu.VMEM((n, dh), jnp.float32)],
    )(adj, x, W1, b, Wn)


def _gcn_layer_final(adj, t, b, Wu, Wv, fcb):
    n = adj.shape[0]
    d = Wu.shape[1]

    def body(adj_ref, t_ref, b_ref, wu_ref, wv_ref, fcb_ref, pu_ref, pv_ref):
        acc = jnp.dot(adj_ref[...], t_ref[...],
                      preferred_element_type=jnp.float32)
        h = jnp.maximum(acc + b_ref[...], 0.0)
        pu = jnp.dot(h, wu_ref[...],
                     preferred_element_type=jnp.float32) + fcb_ref[...]
        pv = jnp.dot(h, wv_ref[...], preferred_element_type=jnp.float32)
        pu_ref[...] = pu
        pv_ref[...] = pv

    return pl.pallas_call(
        body,
        grid=(n // _BM,),
        in_specs=[
            pl.BlockSpec((_BM, n), lambda i: (i, 0)),
            pl.BlockSpec((n, t.shape[1]), lambda i: (0, 0)),
            pl.BlockSpec((1, b.shape[1]), lambda i: (0, 0)),
            pl.BlockSpec(Wu.shape, lambda i: (0, 0)),
            pl.BlockSpec(Wv.shape, lambda i: (0, 0)),
            pl.BlockSpec((1, d), lambda i: (0, 0)),
        ],
        out_specs=[
            pl.BlockSpec((_BM, d), lambda i: (i, 0)),
            pl.BlockSpec((_BM, d), lambda i: (i, 0)),
        ],
        out_shape=[
            jax.ShapeDtypeStruct((n, d), jnp.float32),
            jax.ShapeDtypeStruct((n, d), jnp.float32),
        ],
    )(adj, t, b, Wu, Wv, fcb)


_CHUNK = 80


def _edge_scores_sc(pu, pv, nodes_u, nodes_v):
    e = nodes_u.shape[0]
    d = pu.shape[1]
    info = plsc.get_sparse_core_info()
    nc, ns = info.num_cores, info.num_subcores
    nw = nc * ns
    epw = e // nw
    nch = epw // _CHUNK

    mesh = plsc.VectorSubcoreMesh(core_axis_name="c", subcore_axis_name="s")

    @functools.partial(
        pl.kernel,
        mesh=mesh,
        out_type=jax.ShapeDtypeStruct((e, d), jnp.float32),
        scratch_types=[
            pltpu.VMEM((epw,), jnp.int32),
            pltpu.VMEM((epw,), jnp.int32),
            pltpu.VMEM((_CHUNK, d), jnp.float32),
            pltpu.VMEM((_CHUNK, d), jnp.float32),
            pltpu.VMEM((_CHUNK, d), jnp.float32),
            pltpu.VMEM((_CHUNK, d), jnp.float32),
            pltpu.VMEM((_CHUNK, d), jnp.float32),
            pltpu.VMEM((_CHUNK, d), jnp.float32),
            pltpu.VMEM((_CHUNK, d), jnp.float32),
            pltpu.VMEM((_CHUNK, d), jnp.float32),
            pltpu.VMEM((_CHUNK, d), jnp.float32),
            pltpu.SemaphoreType.DMA,
            pltpu.SemaphoreType.DMA,
            pltpu.SemaphoreType.DMA,
            pltpu.SemaphoreType.DMA,
            pltpu.SemaphoreType.DMA,
            pltpu.SemaphoreType.DMA,
        ],
    )
    def k(pu_hbm, pv_hbm, u_hbm, v_hbm, out_hbm,
          iu_all, iv_all, bu0, bu1, bu2, bv0, bv1, bv2, bo0, bo1, bo2,
          sg0, sg1, sg2, so0, so1, so2):
        wid = lax.axis_index("s") * nc + lax.axis_index("c")
        slots = ((bu0, bv0, bo0, sg0, so0),
                 (bu1, bv1, bo1, sg1, so1),
                 (bu2, bv2, bo2, sg2, so2))

        def base_of(g):
            return pl.multiple_of(wid * epw + g * _CHUNK, 8)

        wbase = pl.multiple_of(wid * epw, 8)
        pltpu.sync_copy(u_hbm.at[pl.ds(wbase, epw)], iu_all)
        pltpu.sync_copy(v_hbm.at[pl.ds(wbase, epw)], iv_all)

        def issue(g, sl):
            bu, bv, _, sg, _ = sl
            off = pl.multiple_of(g * _CHUNK, 8)
            pltpu.async_copy(pu_hbm.at[iu_all.at[pl.ds(off, _CHUNK)]], bu, sg)
            pltpu.async_copy(pv_hbm.at[iv_all.at[pl.ds(off, _CHUNK)]], bv, sg)

        for s in (0, 1, 2):
            issue(s, slots[s])

        def body(i, carry):
            for s in (0, 1, 2):
                g = 3 * i + s
                bu, bv, bo, sg, so = slots[s]

                @pl.when(g < nch)
                def _():
                    pltpu.make_async_copy(pu_hbm.at[iu_all.at[
                        pl.ds(0, _CHUNK)]], bu, sg).wait()
                    pltpu.make_async_copy(pv_hbm.at[iv_all.at[
                        pl.ds(0, _CHUNK)]], bv, sg).wait()

                    @pl.when(g >= 3)
                    def _():
                        pltpu.make_async_copy(
                            bo, out_hbm.at[pl.ds(0, _CHUNK)], so).wait()

                    @plsc.parallel_loop(0, _CHUNK, step=1, unroll=4)
                    def _(r):
                        for j in range(d // 16):
                            sl_ = pl.ds(j * 16, 16)
                            bo[r, sl_] = bu[r, sl_] + bv[r, sl_]
                    pltpu.async_copy(bo, out_hbm.at[pl.ds(base_of(g), _CHUNK)],
                                     so)

                    @pl.when(g + 3 < nch)
                    def _():
                        issue(g + 3, slots[s])
            return carry

        lax.fori_loop(0, (nch + 2) // 3, body, 0)
        for s in (0, 1, 2):
            bo, so = slots[s][2], slots[s][4]
            pltpu.make_async_copy(bo, out_hbm.at[pl.ds(0, _CHUNK)], so).wait()

    return k(pu, pv, nodes_u, nodes_v)


def kernel(x, adj, nodes_u, nodes_v, W1, b1, W2, b2, fc_W, fc_b):
    d = fc_W.shape[1]
    t2 = _gcn_layer1_fused(adj, x, W1, b1.reshape(1, -1), W2)
    pu, pv = _gcn_layer_final(adj, t2, b2.reshape(1, -1),
                              fc_W[:d], fc_W[d:], fc_b.reshape(1, -1))
    return _edge_scores_sc(pu, pv, nodes_u, nodes_v)


# speedup vs baseline: 1.0187x; 1.0187x over previous
import functools

import jax
import jax.numpy as jnp
from jax import lax
from jax.experimental import pallas as pl
from jax.experimental.pallas import tpu as pltpu
from jax.experimental.pallas import tpu_sc as plsc


_BM = 400


def _gcn_two_layers(adj, x, W1, b1, W2, b2, Wu, Wv, fcb):
    n = adj.shape[0]
    d = Wu.shape[1]
    nb = n // _BM

    def body(adj_ref, x_ref, w1_ref, b1_ref, w2_ref, b2_ref,
             wu_ref, wv_ref, fcb_ref, pu_ref, pv_ref, t1_scr, t2_scr):
        i = pl.program_id(0)

        @pl.when(i == 0)
        def _():
            t1_scr[...] = jnp.dot(x_ref[...], w1_ref[...],
                                  preferred_element_type=jnp.float32)

        @pl.when(i < nb)
        def _():
            acc = jnp.dot(adj_ref[...], t1_scr[...],
                          preferred_element_type=jnp.float32)
            h1 = jnp.maximum(acc + b1_ref[...], 0.0)
            row = pl.multiple_of(i * _BM, _BM)
            t2_scr[pl.ds(row, _BM), :] = jnp.dot(
                h1, w2_ref[...], preferred_element_type=jnp.float32)

        @pl.when(i >= nb)
        def _():
            acc = jnp.dot(adj_ref[...], t2_scr[...],
                          preferred_element_type=jnp.float32)
            h2 = jnp.maximum(acc + b2_ref[...], 0.0)
            pu_ref[...] = jnp.dot(h2, wu_ref[...],
                                  preferred_element_type=jnp.float32
                                  ) + fcb_ref[...]
            pv_ref[...] = jnp.dot(h2, wv_ref[...],
                                  preferred_element_type=jnp.float32)

    out_map = lambda i: (jnp.maximum(i - nb, 0), 0)
    return pl.pallas_call(
        body,
        grid=(2 * nb,),
        in_specs=[
            pl.BlockSpec((_BM, n), lambda i: (lax.rem(i, nb), 0)),
            pl.BlockSpec(x.shape, lambda i: (0, 0)),
            pl.BlockSpec(W1.shape, lambda i: (0, 0)),
            pl.BlockSpec((1, b1.shape[1]), lambda i: (0, 0)),
            pl.BlockSpec(W2.shape, lambda i: (0, 0)),
            pl.BlockSpec((1, b2.shape[1]), lambda i: (0, 0)),
            pl.BlockSpec(Wu.shape, lambda i: (0, 0)),
            pl.BlockSpec(Wv.shape, lambda i: (0, 0)),
            pl.BlockSpec((1, d), lambda i: (0, 0)),
        ],
        out_specs=[
            pl.BlockSpec((_BM, d), out_map),
            pl.BlockSpec((_BM, d), out_map),
        ],
        out_shape=[
            jax.ShapeDtypeStruct((n, d), jnp.float32),
            jax.ShapeDtypeStruct((n, d), jnp.float32),
        ],
        scratch_shapes=[
            pltpu.VMEM((n, W1.shape[1]), jnp.float32),
            pltpu.VMEM((n, W2.shape[1]), jnp.float32),
        ],
    )(adj, x, W1, b1, W2, b2, Wu, Wv, fcb)


_CHUNK = 80


def _edge_scores_sc(pu, pv, nodes_u, nodes_v):
    e = nodes_u.shape[0]
    d = pu.shape[1]
    info = plsc.get_sparse_core_info()
    nc, ns = info.num_cores, info.num_subcores
    nw = nc * ns
    epw = e // nw
    nch = epw // _CHUNK

    mesh = plsc.VectorSubcoreMesh(core_axis_name="c", subcore_axis_name="s")

    @functools.partial(
        pl.kernel,
        mesh=mesh,
        out_type=jax.ShapeDtypeStruct((e, d), jnp.float32),
        scratch_types=[
            pltpu.VMEM((epw,), jnp.int32),
            pltpu.VMEM((epw,), jnp.int32),
            pltpu.VMEM((_CHUNK, d), jnp.float32),
            pltpu.VMEM((_CHUNK, d), jnp.float32),
            pltpu.VMEM((_CHUNK, d), jnp.float32),
            pltpu.VMEM((_CHUNK, d), jnp.float32),
            pltpu.VMEM((_CHUNK, d), jnp.float32),
            pltpu.VMEM((_CHUNK, d), jnp.float32),
            pltpu.VMEM((_CHUNK, d), jnp.float32),
            pltpu.VMEM((_CHUNK, d), jnp.float32),
            pltpu.VMEM((_CHUNK, d), jnp.float32),
            pltpu.SemaphoreType.DMA,
            pltpu.SemaphoreType.DMA,
            pltpu.SemaphoreType.DMA,
            pltpu.SemaphoreType.DMA,
            pltpu.SemaphoreType.DMA,
            pltpu.SemaphoreType.DMA,
        ],
    )
    def k(pu_hbm, pv_hbm, u_hbm, v_hbm, out_hbm,
          iu_all, iv_all, bu0, bu1, bu2, bv0, bv1, bv2, bo0, bo1, bo2,
          sg0, sg1, sg2, so0, so1, so2):
        wid = lax.axis_index("s") * nc + lax.axis_index("c")
        slots = ((bu0, bv0, bo0, sg0, so0),
                 (bu1, bv1, bo1, sg1, so1),
                 (bu2, bv2, bo2, sg2, so2))

        def base_of(g):
            return pl.multiple_of(wid * epw + g * _CHUNK, 8)

        wbase = pl.multiple_of(wid * epw, 8)
        pltpu.sync_copy(u_hbm.at[pl.ds(wbase, epw)], iu_all)
        pltpu.sync_copy(v_hbm.at[pl.ds(wbase, epw)], iv_all)

        def issue(g, sl):
            bu, bv, _, sg, _ = sl
            off = pl.multiple_of(g * _CHUNK, 8)
            pltpu.async_copy(pu_hbm.at[iu_all.at[pl.ds(off, _CHUNK)]], bu, sg)
            pltpu.async_copy(pv_hbm.at[iv_all.at[pl.ds(off, _CHUNK)]], bv, sg)

        for s in (0, 1, 2):
            issue(s, slots[s])

        def body(i, carry):
            for s in (0, 1, 2):
                g = 3 * i + s
                bu, bv, bo, sg, so = slots[s]

                @pl.when(g < nch)
                def _():
                    pltpu.make_async_copy(pu_hbm.at[iu_all.at[
                        pl.ds(0, _CHUNK)]], bu, sg).wait()
                    pltpu.make_async_copy(pv_hbm.at[iv_all.at[
                        pl.ds(0, _CHUNK)]], bv, sg).wait()

                    @pl.when(g >= 3)
                    def _():
                        pltpu.make_async_copy(
                            bo, out_hbm.at[pl.ds(0, _CHUNK)], so).wait()

                    @plsc.parallel_loop(0, _CHUNK, step=1, unroll=4)
                    def _(r):
                        for j in range(d // 16):
                            sl_ = pl.ds(j * 16, 16)
                            bo[r, sl_] = bu[r, sl_] + bv[r, sl_]
                    pltpu.async_copy(bo, out_hbm.at[pl.ds(base_of(g), _CHUNK)],
                                     so)

                    @pl.when(g + 3 < nch)
                    def _():
                        issue(g + 3, slots[s])
            return carry

        lax.fori_loop(0, (nch + 2) // 3, body, 0)
        for s in (0, 1, 2):
            bo, so = slots[s][2], slots[s][4]
            pltpu.make_async_copy(bo, out_hbm.at[pl.ds(0, _CHUNK)], so).wait()

    return k(pu, pv, nodes_u, nodes_v)


def kernel(x, adj, nodes_u, nodes_v, W1, b1, W2, b2, fc_W, fc_b):
    d = fc_W.shape[1]
    pu, pv = _gcn_two_layers(adj, x, W1, b1.reshape(1, -1), W2,
                             b2.reshape(1, -1), fc_W[:d], fc_W[d:],
                             fc_b.reshape(1, -1))
    return _edge_scores_sc(pu, pv, nodes_u, nodes_v)
